# baseline (speedup 1.0000x reference)
import functools

import jax
import jax.numpy as jnp
from jax import lax
from jax.experimental import pallas as pl
from jax.experimental.pallas import tpu as pltpu
from jax.experimental.pallas import tpu_sc as plsc

N = 50000
E = 800000
D = 64
HALF = N // 2
G = 128
EP = 819200
EPT = EP // 16
CE = 6400
CI = CE // G
NOUT = EPT // CE
TRASH = 25088
ACC_R = 25096
ZROWS = 1568
ZLAST = HALF - 15 * ZROWS
NBUF = 3

_mesh = plsc.VectorSubcoreMesh(core_axis_name="c", subcore_axis_name="s")


@functools.partial(
    pl.kernel,
    out_type=jax.ShapeDtypeStruct((N, 8), jnp.float32),
    mesh=_mesh,
    compiler_params=pltpu.CompilerParams(use_tc_tiling_on_sc=False),
    scratch_types=[
        pltpu.VMEM((2, CI, G), jnp.int32),
        pltpu.VMEM((G, 8), jnp.float32),
        pltpu.VMEM_SHARED((ACC_R, 8), jnp.float32),
        pltpu.SemaphoreType.DMA,
        pltpu.SemaphoreType.DMA,
    ],
)
def _degcount(dst2, zb8, ones8, deg8, dstb, onesb, acc8, dsem, stsem):
    c = lax.axis_index("c")
    s = lax.axis_index("s")
    lo = c * HALF
    pltpu.sync_copy(ones8, onesb)

    @pl.when(s < 15)
    def _zero_main():
        pltpu.sync_copy(zb8, acc8.at[pl.ds(s * ZROWS, ZROWS)])

    @pl.when(s == 15)
    def _zero_last():
        pltpu.sync_copy(zb8.at[pl.ds(0, ZLAST)],
                        acc8.at[pl.ds(15 * ZROWS, ZLAST)])

    plsc.subcore_barrier()

    base = s * EPT

    def _stage(i, ib):
        pltpu.async_copy(dst2.at[pl.ds((base + i * CE) // G, CI)],
                         dstb.at[ib], stsem)

    _stage(0, 0)

    def outer(i, carry):
        ib = lax.rem(i, 2)
        pltpu.make_async_copy(dst2.at[pl.ds(0, CI)], dstb.at[ib],
                              stsem).wait()

        @pl.when(i + 1 < NOUT)
        def _stage_next():
            _stage(i + 1, 1 - ib)

        def comp(j, carry2):
            for k in range(G // 16):
                d = dstb[ib, j, pl.ds(k * 16, 16)]
                keep = (d >= lo) & (d < lo + HALF)
                dstb[ib, j, pl.ds(k * 16, 16)] = jnp.where(keep, d - lo, TRASH)
            return carry2

        lax.fori_loop(0, CI, comp, 0)

        for j in range(CI):
            pltpu.async_copy(onesb, acc8.at[dstb.at[ib, j]], dsem, add=True)
            if j >= 6:
                pltpu.make_async_copy(onesb, acc8.at[dstb.at[ib, 0]],
                                      dsem).wait()
        for j in range(min(6, CI)):
            pltpu.make_async_copy(onesb, acc8.at[dstb.at[ib, 0]],
                                  dsem).wait()
        return carry

    lax.fori_loop(0, NOUT, outer, 0)
    plsc.subcore_barrier()

    @pl.when(s < 15)
    def _out_main():
        pltpu.sync_copy(acc8.at[pl.ds(s * ZROWS, ZROWS)],
                        deg8.at[pl.ds(c * HALF + s * ZROWS, ZROWS)])

    @pl.when(s == 15)
    def _out_last():
        pltpu.sync_copy(acc8.at[pl.ds(15 * ZROWS, ZLAST)],
                        deg8.at[pl.ds(c * HALF + 15 * ZROWS, ZLAST)])


SCE = 1024
SCI = SCE // G
NGRP = EPT // G
NCH = EPT // SCE


@functools.partial(
    pl.kernel,
    out_type=jax.ShapeDtypeStruct((N, D), jnp.float32),
    mesh=_mesh,
    compiler_params=pltpu.CompilerParams(use_tc_tiling_on_sc=False),
    scratch_types=[
        pltpu.VMEM((2, SCE), jnp.int32),
        pltpu.VMEM((2, SCI, G), jnp.int32),
        pltpu.VMEM((NBUF, G, D), jnp.float32),
        pltpu.VMEM_SHARED((ACC_R, D), jnp.float32),
        pltpu.SemaphoreType.DMA((NBUF,)),
        pltpu.SemaphoreType.DMA((NBUF,)),
        pltpu.SemaphoreType.DMA,
    ],
)
def _segsum(h, src, dst2, zb, seg, srcb, dstb, rows, acc, gsem, ssem, stsem):
    c = lax.axis_index("c")
    s = lax.axis_index("s")
    lo = c * HALF

    @pl.when(s < 15)
    def _zero_main():
        pltpu.sync_copy(zb, acc.at[pl.ds(s * ZROWS, ZROWS)])

    @pl.when(s == 15)
    def _zero_last():
        pltpu.sync_copy(zb.at[pl.ds(0, ZLAST)],
                        acc.at[pl.ds(15 * ZROWS, ZLAST)])

    plsc.subcore_barrier()

    base = s * EPT

    def _stage(i, ib):
        e0 = base + i * SCE
        pltpu.async_copy(src.at[pl.ds(e0, SCE)], srcb.at[ib], stsem)
        pltpu.async_copy(dst2.at[pl.ds(e0 // G, SCI)], dstb.at[ib], stsem)

    def _drain_stage():
        pltpu.make_async_copy(src.at[pl.ds(0, SCE)], srcb.at[0],
                              stsem).wait()
        pltpu.make_async_copy(dst2.at[pl.ds(0, SCI)], dstb.at[0],
                              stsem).wait()

    def _issue_gather(gn):
        cn = lax.div(gn, SCI)
        pltpu.async_copy(
            h.at[srcb.at[lax.rem(cn, 2), pl.ds(lax.rem(gn, SCI) * G, G)]],
            rows.at[lax.rem(gn, NBUF)], gsem.at[lax.rem(gn, NBUF)])

    _stage(0, 0)
    _drain_stage()
    for g in range(NBUF - 1):
        _issue_gather(g)

    def grp(g, carry):
        ch = lax.div(g, SCI)
        jj = lax.rem(g, SCI)
        ib = lax.rem(ch, 2)
        b = lax.rem(g, NBUF)

        for k in range(G // 16):
            d = dstb[ib, jj, pl.ds(k * 16, 16)]
            keep = (d >= lo) & (d < lo + HALF)
            dstb[ib, jj, pl.ds(k * 16, 16)] = jnp.where(keep, d - lo, TRASH)

        pltpu.make_async_copy(h.at[srcb.at[0, pl.ds(0, G)]], rows.at[b],
                              gsem.at[b]).wait()

        @pl.when(g >= 1)
        def _wait_prev_scatter():
            bp = lax.rem(g + NBUF - 1, NBUF)
            pltpu.make_async_copy(rows.at[bp], acc.at[dstb.at[0, 0]],
                                  ssem.at[bp]).wait()

        @pl.when((jj == 0) & (ch + 1 < NCH))
        def _issue_stage():
            _stage(ch + 1, 1 - ib)

        @pl.when((jj == SCI - NBUF + 1) & (ch + 1 < NCH))
        def _drain_stage_next():
            _drain_stage()

        @pl.when(g + NBUF - 1 < NGRP)
        def _issue_next():
            _issue_gather(g + NBUF - 1)

        pltpu.async_copy(rows.at[b], acc.at[dstb.at[ib, jj]], ssem.at[b],
                         add=True)
        return carry

    lax.fori_loop(0, NGRP, grp, 0)
    bl = (NGRP - 1) % NBUF
    pltpu.make_async_copy(rows.at[bl], acc.at[dstb.at[0, 0]],
                          ssem.at[bl]).wait()
    plsc.subcore_barrier()

    @pl.when(s < 15)
    def _out_main():
        pltpu.sync_copy(acc.at[pl.ds(s * ZROWS, ZROWS)],
                        seg.at[pl.ds(c * HALF + s * ZROWS, ZROWS)])

    @pl.when(s == 15)
    def _out_last():
        pltpu.sync_copy(acc.at[pl.ds(15 * ZROWS, ZLAST)],
                        seg.at[pl.ds(c * HALF + 15 * ZROWS, ZLAST)])


BN = 2000


def _embed_body(x_ref, w_ref, b_ref, o_ref):
    o_ref[...] = (
        jnp.dot(x_ref[...], w_ref[...], preferred_element_type=jnp.float32)
        + b_ref[...]
    )


def _embed(x, W_in, b_in):
    return pl.pallas_call(
        _embed_body,
        grid=(N // BN,),
        in_specs=[
            pl.BlockSpec((BN, 2), lambda i: (i, 0)),
            pl.BlockSpec((2, D), lambda i: (0, 0)),
            pl.BlockSpec((1, D), lambda i: (0, 0)),
        ],
        out_specs=pl.BlockSpec((BN, D), lambda i: (i, 0)),
        out_shape=jax.ShapeDtypeStruct((N, D), jnp.float32),
    )(x, W_in, b_in)


def _dense_body(h_ref, seg_ref, deg_ref, ws_ref, wn_ref, b_ref, o_ref):
    h = h_ref[...]
    deg = jnp.maximum(deg_ref[:, 0:1], 1.0)
    msg = seg_ref[...] / deg
    z = (
        jnp.dot(h, ws_ref[...], preferred_element_type=jnp.float32)
        + jnp.dot(msg, wn_ref[...], preferred_element_type=jnp.float32)
        + b_ref[...]
    )
    o_ref[...] = h + jnp.where(z >= 0, z, 0.01 * z)


def _dense(h, seg, deg8, Ws, Wn, bias):
    return pl.pallas_call(
        _dense_body,
        grid=(N // BN,),
        in_specs=[
            pl.BlockSpec((BN, D), lambda i: (i, 0)),
            pl.BlockSpec((BN, D), lambda i: (i, 0)),
            pl.BlockSpec((BN, 8), lambda i: (i, 0)),
            pl.BlockSpec((D, D), lambda i: (0, 0)),
            pl.BlockSpec((D, D), lambda i: (0, 0)),
            pl.BlockSpec((1, D), lambda i: (0, 0)),
        ],
        out_specs=pl.BlockSpec((BN, D), lambda i: (i, 0)),
        out_shape=jax.ShapeDtypeStruct((N, D), jnp.float32),
    )(h, seg, deg8, Ws, Wn, bias)


def kernel(x, edge_index, W_in, b_in, Wself, Wnei, b):
    pad = EP - E
    src = jnp.concatenate([edge_index[0], jnp.zeros((pad,), jnp.int32)])
    dst = jnp.concatenate([edge_index[1], jnp.full((pad,), -1, jnp.int32)])
    dst2 = dst.reshape(EP // G, G)
    zb = jnp.zeros((ZROWS, D), jnp.float32)
    zb8 = jnp.zeros((ZROWS, 8), jnp.float32)
    ones8 = jnp.ones((G, 8), jnp.float32)

    h = _embed(x, W_in, b_in.reshape(1, D))
    deg8 = _degcount(dst2, zb8, ones8)
    for l in range(3):
        seg = _segsum(h, src, dst2, zb)
        h = _dense(h, seg, deg8, Wself[l], Wnei[l], b[l].reshape(1, D))
    return h
